# Initial kernel scaffold; baseline (speedup 1.0000x reference)
import jax
import jax.numpy as jnp
from jax.experimental import pallas as pl


def kernel(x, edge_index, edge_weights, W0, b0, hop1, W1, b1, hop2, W2, b2, Wh1, bh1, Wh2, bh2):
    raise NotImplementedError("write your pallas kernel here")


# speedup vs baseline: 3.7410x; 3.7410x over previous
import functools

import jax
import jax.numpy as jnp
from jax import lax
from jax.experimental import pallas as pl
from jax.experimental.pallas import tpu as pltpu
from jax.experimental.pallas import tpu_sc as plsc

N = 10000
E = 320000
D = 128
K = 5
C = 64

BR = 400
NB = N // BR
NC, NS = 2, 16
NW = NC * NS
B = 128
NFULL = 79
EPW = NFULL * B
EPAD = NW * EPW
NROWS = N + 16
RPT = 624
ZR = 80


def _mlp_body(x_ref, w_ref, b_ref, o_ref):
    o_ref[...] = jnp.maximum(
        jnp.dot(x_ref[...], w_ref[...], preferred_element_type=jnp.float32)
        + b_ref[...], 0.0)


_mlp = pl.pallas_call(
    _mlp_body,
    grid=(NB,),
    in_specs=[pl.BlockSpec((BR, D), lambda i: (i, 0)),
              pl.BlockSpec((D, D), lambda i: (0, 0)),
              pl.BlockSpec((1, D), lambda i: (0, 0))],
    out_specs=pl.BlockSpec((BR, D), lambda i: (i, 0)),
    out_shape=jax.ShapeDtypeStruct((N, D), jnp.float32),
)


def _combine_body(h_ref, a0_ref, a1_ref, w_ref, b_ref, o_ref):
    s = h_ref[...] + a0_ref[...] + a1_ref[...]
    o_ref[...] = jnp.maximum(
        jnp.dot(s, w_ref[...], preferred_element_type=jnp.float32)
        + b_ref[...], 0.0)


_combine = pl.pallas_call(
    _combine_body,
    grid=(NB,),
    in_specs=[pl.BlockSpec((BR, D), lambda i: (i, 0)),
              pl.BlockSpec((BR, D), lambda i: (i, 0)),
              pl.BlockSpec((BR, D), lambda i: (i, 0)),
              pl.BlockSpec((D, D), lambda i: (0, 0)),
              pl.BlockSpec((1, D), lambda i: (0, 0))],
    out_specs=pl.BlockSpec((BR, D), lambda i: (i, 0)),
    out_shape=jax.ShapeDtypeStruct((N, D), jnp.float32),
)


def _scale_body(hop_ref, h_ref, o_ref):
    hrow = hop_ref[...]
    m = jnp.max(hrow)
    e = jnp.exp(hrow - m)
    w = e / jnp.sum(e)
    hb = h_ref[...]
    for kk in range(K):
        o_ref[kk] = hb * w[0, kk]


_scale = pl.pallas_call(
    _scale_body,
    grid=(NB,),
    in_specs=[pl.BlockSpec((1, K), lambda i: (0, 0)),
              pl.BlockSpec((BR, D), lambda i: (i, 0))],
    out_specs=pl.BlockSpec((K, BR, D), lambda i: (0, i, 0)),
    out_shape=jax.ShapeDtypeStruct((K, N, D), jnp.float32),
)


def _head_body(h_ref, w1_ref, b1_ref, w2_ref, b2_ref, o_ref):
    t = jnp.maximum(
        jnp.dot(h_ref[...], w1_ref[...], preferred_element_type=jnp.float32)
        + b1_ref[...], 0.0)
    o_ref[...] = (jnp.dot(t, w2_ref[...], preferred_element_type=jnp.float32)
                  + b2_ref[...])


_head = pl.pallas_call(
    _head_body,
    grid=(NB,),
    in_specs=[pl.BlockSpec((BR, D), lambda i: (i, 0)),
              pl.BlockSpec((D, D), lambda i: (0, 0)),
              pl.BlockSpec((1, D), lambda i: (0, 0)),
              pl.BlockSpec((D, C), lambda i: (0, 0)),
              pl.BlockSpec((1, C), lambda i: (0, 0))],
    out_specs=pl.BlockSpec((BR, C), lambda i: (i, 0)),
    out_shape=jax.ShapeDtypeStruct((N, C), jnp.float32),
)


def _gidx_body(src_ref, ew_ref, o_ref):
    o_ref[...] = ew_ref[...] * N + src_ref[...]


_gidx = pl.pallas_call(
    _gidx_body,
    out_shape=jax.ShapeDtypeStruct((EPAD // 128, 128), jnp.int32),
)


_mesh = plsc.VectorSubcoreMesh(core_axis_name="c", subcore_axis_name="s")


@functools.partial(
    pl.kernel,
    out_type=jax.ShapeDtypeStruct((NC, N, D), jnp.float32),
    mesh=_mesh,
    scratch_types=[
        pltpu.VMEM((B,), jnp.int32),
        pltpu.VMEM((B,), jnp.int32),
        pltpu.VMEM((B, D), jnp.float32),
        pltpu.VMEM((ZR, D), jnp.float32),
        pltpu.VMEM_SHARED((NROWS, D), jnp.float32),
        pltpu.SemaphoreType.DMA,
    ],
)
def _sc_agg(scaled_hbm, gidx_hbm, dst_hbm, out_hbm,
            gi_v, di_v, rows_v, zbuf_v, acc_sh, sem):
    cid = lax.axis_index("c")
    sid = lax.axis_index("s")
    wid = cid * NS + sid

    zv = jnp.zeros((16,), jnp.float32)

    def _zb(i, carry):
        zbuf_v[i // 8, pl.ds((i % 8) * 16, 16)] = zv
        return carry

    lax.fori_loop(0, ZR * 8, _zb, 0)
    r0 = sid * RPT
    nz = RPT // ZR
    for j in range(nz):
        pltpu.sync_copy(zbuf_v, acc_sh.at[pl.ds(r0 + j * ZR, ZR)])
    rem = RPT - nz * ZR
    pltpu.sync_copy(zbuf_v.at[pl.ds(0, rem)],
                    acc_sh.at[pl.ds(r0 + nz * ZR, rem)])

    @pl.when(sid == 0)
    def _zero_tail():
        pltpu.sync_copy(zbuf_v.at[pl.ds(0, 16)],
                        acc_sh.at[pl.ds(NS * RPT, 16)])

    plsc.subcore_barrier()

    base = wid * EPW

    def _chunk(ci, carry):
        off = base + ci * B
        pltpu.sync_copy(gidx_hbm.at[pl.ds(off, B)], gi_v)
        pltpu.sync_copy(dst_hbm.at[pl.ds(off, B)], di_v)
        pltpu.async_copy(scaled_hbm.at[gi_v], rows_v, sem).wait()
        pltpu.sync_copy(rows_v, acc_sh.at[di_v], add=True)
        return carry

    lax.fori_loop(0, NFULL, _chunk, 0)

    plsc.subcore_barrier()
    pltpu.sync_copy(acc_sh.at[pl.ds(r0, RPT)],
                    out_hbm.at[cid, pl.ds(r0, RPT)])

    @pl.when(sid == 0)
    def _flush_tail():
        pltpu.sync_copy(acc_sh.at[pl.ds(NS * RPT, 16)],
                        out_hbm.at[cid, pl.ds(NS * RPT, 16)])


def kernel(x, edge_index, edge_weights, W0, b0, hop1, W1, b1,
           hop2, W2, b2, Wh1, bh1, Wh2, bh2):
    src = edge_index[0]
    dst = edge_index[1]
    pad = EPAD - E
    srcp = jnp.concatenate([src, jnp.zeros((pad,), jnp.int32)])
    ewp = jnp.concatenate([edge_weights, jnp.zeros((pad,), jnp.int32)])
    dstp = jnp.concatenate([dst, jnp.full((pad,), N, jnp.int32)])
    gidx = _gidx(srcp.reshape(EPAD // 128, 128),
                 ewp.reshape(EPAD // 128, 128)).reshape(EPAD)

    b0r = b0.reshape(1, D)
    h = _mlp(x, W0, b0r)
    for hop, W, b in ((hop1, W1, b1), (hop2, W2, b2)):
        s = _scale(hop.reshape(1, K), h).reshape(K * N, D)
        p = _sc_agg(s, gidx, dstp)
        h = _combine(h, p[0], p[1], W, b.reshape(1, D))
    out = _head(h, Wh1, bh1.reshape(1, D), Wh2, bh2.reshape(1, C))
    return out
